# baseline (speedup 1.0000x reference)
import functools

import jax
import jax.numpy as jnp
from jax import lax
from jax.experimental import pallas as pl
from jax.experimental.pallas import tpu as pltpu
from jax.experimental.pallas import tpu_sc as plsc

NUM_CORES = 2
NUM_SUBCORES = 16
LANES = 16
BLK = 128


def _make_kernel(batch, seq, vocab, dim):
    dpad = (dim + BLK - 1) // BLK * BLK
    spad = (seq + 7) // 8 * 8
    n_bb = batch // BLK
    n_vb = dpad // BLK
    assert n_bb * n_vb == 64
    tail = dim - (n_vb - 1) * BLK
    n_items = 2 * seq

    mesh = plsc.VectorSubcoreMesh(core_axis_name="c", subcore_axis_name="s")

    @functools.partial(
        pl.kernel,
        mesh=mesh,
        compiler_params=pltpu.CompilerParams(
            use_tc_tiling_on_sc=True, needs_layout_passes=False),
        out_type=jax.ShapeDtypeStruct((seq, dim, batch), jnp.float32),
        scratch_types=[
            pltpu.VMEM((spad, BLK), jnp.int32),
            pltpu.VMEM((BLK, BLK), jnp.float32),
            pltpu.VMEM((BLK, BLK), jnp.float32),
            pltpu.VMEM((BLK, BLK), jnp.float32),
            pltpu.VMEM((BLK, BLK), jnp.float32),
            pltpu.SemaphoreType.DMA,
            pltpu.SemaphoreType.DMA,
            pltpu.SemaphoreType.DMA,
            pltpu.SemaphoreType.DMA,
        ],
    )
    def k(table_hbm, idxT_hbm, out_hbm, idx_v, G0, G1, X0, X1, g0, g1, s0, s1):
        wid = lax.axis_index("s") * NUM_CORES + lax.axis_index("c")
        bb = wid // 4
        vb0 = 2 * (wid % 4)
        bcol = pl.multiple_of(bb * BLK, BLK)
        G = (G0, G1)
        XT = (X0, X1)
        gsem = (g0, g1)
        ssem = (s0, s1)

        pltpu.sync_copy(idxT_hbm.at[:, pl.ds(bcol, BLK)], idx_v)

        def tv(i):
            return vb0 + i // seq, i % seq

        def g_start(i, b):
            vb, t = tv(i)
            vcol = pl.multiple_of(vb * BLK, BLK)
            pltpu.async_copy(
                table_hbm.at[idx_v.at[t], pl.ds(vcol, BLK)], G[b], gsem[b])

        def g_wait(b):
            pltpu.make_async_copy(
                table_hbm.at[pl.ds(0, BLK), pl.ds(0, BLK)], G[b], gsem[b]).wait()

        def s_start(i, b):
            vb, t = tv(i)
            last = vb == (n_vb - 1)

            @pl.when(last)
            def _():
                pltpu.async_copy(
                    XT[b].at[pl.ds(0, tail)],
                    out_hbm.at[t, pl.ds((n_vb - 1) * BLK, tail), pl.ds(bcol, BLK)],
                    ssem[b])

            @pl.when(jnp.logical_not(last))
            def _():
                vcol = pl.multiple_of(vb * BLK, BLK)
                pltpu.async_copy(
                    XT[b], out_hbm.at[t, pl.ds(vcol, BLK), pl.ds(bcol, BLK)],
                    ssem[b])

        def s_wait(i, b):
            vb, _ = tv(i)
            last = vb == (n_vb - 1)

            @pl.when(last)
            def _():
                pltpu.make_async_copy(
                    XT[b].at[pl.ds(0, tail)],
                    out_hbm.at[0, pl.ds(0, tail), pl.ds(0, BLK)], ssem[b]).wait()

            @pl.when(jnp.logical_not(last))
            def _():
                pltpu.make_async_copy(
                    XT[b], out_hbm.at[0, pl.ds(0, BLK), pl.ds(0, BLK)],
                    ssem[b]).wait()

        def transpose(b):
            src = G[b]
            dst = XT[b]
            for j0 in range(0, BLK, LANES):
                rows = lax.iota(jnp.int32, LANES) + j0

                @pl.loop(0, BLK, unroll=8)
                def _(v):
                    cols = jnp.full((LANES,), v, jnp.int32)
                    dst[v, pl.ds(j0, LANES)] = plsc.load_gather(src, [rows, cols])

        g_start(0, 0)
        g_wait(0)
        g_start(1, 1)
        transpose(0)
        s_start(0, 0)
        g_wait(1)
        g_start(2, 0)
        transpose(1)
        s_start(1, 1)

        @pl.loop(2, n_items - 2, step=2)
        def _(i):
            g_wait(0)
            s_wait(i - 2, 0)
            g_start(i + 1, 1)
            transpose(0)
            s_start(i, 0)
            g_wait(1)
            s_wait(i - 1, 1)
            g_start(i + 2, 0)
            transpose(1)
            s_start(i + 1, 1)

        g_wait(0)
        s_wait(n_items - 4, 0)
        g_start(n_items - 1, 1)
        transpose(0)
        s_start(n_items - 2, 0)
        g_wait(1)
        s_wait(n_items - 3, 1)
        transpose(1)
        s_start(n_items - 1, 1)
        s_wait(n_items - 2, 0)
        s_wait(n_items - 1, 1)

    return k


def kernel(token_indices, token_embedding_table):
    b, t = token_indices.shape
    v, d = token_embedding_table.shape
    idx_t = jnp.pad(token_indices.astype(jnp.int32).T, ((0, 6), (0, 0)))
    table_pad = jnp.pad(token_embedding_table, ((0, 0), (0, 24)))
    x = _make_kernel(b, t, v, d)(table_pad, idx_t)
    return jnp.transpose(x, (2, 0, 1))


# speedup vs baseline: 1.7526x; 1.7526x over previous
import functools

import jax
import jax.numpy as jnp
from jax import lax
from jax.experimental import pallas as pl
from jax.experimental.pallas import tpu as pltpu
from jax.experimental.pallas import tpu_sc as plsc

NUM_CORES = 2
NUM_SUBCORES = 16
LANES = 16
BLK = 128


def _make_kernel(batch, seq, vocab, dim):
    dpad = (dim + BLK - 1) // BLK * BLK
    spad = (seq + 7) // 8 * 8
    n_bb = batch // BLK
    n_vb = dpad // BLK
    assert n_bb * n_vb == 64
    tail = dim - (n_vb - 1) * BLK
    n_items = 2 * seq

    mesh = plsc.VectorSubcoreMesh(core_axis_name="c", subcore_axis_name="s")

    @functools.partial(
        pl.kernel,
        mesh=mesh,
        compiler_params=pltpu.CompilerParams(
            use_tc_tiling_on_sc=True, needs_layout_passes=False),
        out_type=jax.ShapeDtypeStruct((seq, dim, batch), jnp.float32),
        scratch_types=[
            pltpu.VMEM((spad, BLK), jnp.int32),
            pltpu.VMEM((BLK, BLK), jnp.float32),
            pltpu.VMEM((BLK, BLK), jnp.float32),
            pltpu.VMEM((BLK, BLK), jnp.float32),
            pltpu.VMEM((BLK, BLK), jnp.float32),
            pltpu.SemaphoreType.DMA,
            pltpu.SemaphoreType.DMA,
            pltpu.SemaphoreType.DMA,
            pltpu.SemaphoreType.DMA,
        ],
    )
    def k(table_hbm, idxT_hbm, out_hbm, idx_v, G0, G1, X0, X1, g0, g1, s0, s1):
        wid = lax.axis_index("s") * NUM_CORES + lax.axis_index("c")
        bb = wid // 4
        vb0 = 2 * (wid % 4)
        bcol = pl.multiple_of(bb * BLK, BLK)
        G = (G0, G1)
        XT = (X0, X1)
        gsem = (g0, g1)
        ssem = (s0, s1)

        pltpu.sync_copy(idxT_hbm.at[:, pl.ds(bcol, BLK)], idx_v)

        def tv(i):
            return vb0 + i // seq, i % seq

        def g_start(i, b):
            vb, t = tv(i)
            vcol = pl.multiple_of(vb * BLK, BLK)
            pltpu.async_copy(
                table_hbm.at[idx_v.at[t], pl.ds(vcol, BLK)], G[b], gsem[b])

        def g_wait(b):
            pltpu.make_async_copy(
                table_hbm.at[pl.ds(0, BLK), pl.ds(0, BLK)], G[b], gsem[b]).wait()

        def s_start(i, b):
            vb, t = tv(i)
            last = vb == (n_vb - 1)

            @pl.when(last)
            def _():
                pltpu.async_copy(
                    XT[b].at[pl.ds(0, tail)],
                    out_hbm.at[t, pl.ds((n_vb - 1) * BLK, tail), pl.ds(bcol, BLK)],
                    ssem[b])

            @pl.when(jnp.logical_not(last))
            def _():
                vcol = pl.multiple_of(vb * BLK, BLK)
                pltpu.async_copy(
                    XT[b], out_hbm.at[t, pl.ds(vcol, BLK), pl.ds(bcol, BLK)],
                    ssem[b])

        def s_wait(i, b):
            vb, _ = tv(i)
            last = vb == (n_vb - 1)

            @pl.when(last)
            def _():
                pltpu.make_async_copy(
                    XT[b].at[pl.ds(0, tail)],
                    out_hbm.at[0, pl.ds(0, tail), pl.ds(0, BLK)], ssem[b]).wait()

            @pl.when(jnp.logical_not(last))
            def _():
                pltpu.make_async_copy(
                    XT[b], out_hbm.at[0, pl.ds(0, BLK), pl.ds(0, BLK)],
                    ssem[b]).wait()

        def transpose(b):
            src = G[b]
            dst = XT[b]
            for j0 in range(0, BLK, LANES):
                rows = lax.iota(jnp.int32, LANES) + j0

                @plsc.parallel_loop(0, BLK, unroll=8)
                def _(v):
                    cols = jnp.full((LANES,), v, jnp.int32)
                    dst[v, pl.ds(j0, LANES)] = plsc.load_gather(src, [rows, cols])

        g_start(0, 0)
        g_wait(0)
        g_start(1, 1)
        transpose(0)
        s_start(0, 0)
        g_wait(1)
        g_start(2, 0)
        transpose(1)
        s_start(1, 1)

        @pl.loop(2, n_items - 2, step=2)
        def _(i):
            g_wait(0)
            s_wait(i - 2, 0)
            g_start(i + 1, 1)
            transpose(0)
            s_start(i, 0)
            g_wait(1)
            s_wait(i - 1, 1)
            g_start(i + 2, 0)
            transpose(1)
            s_start(i + 1, 1)

        g_wait(0)
        s_wait(n_items - 4, 0)
        g_start(n_items - 1, 1)
        transpose(0)
        s_start(n_items - 2, 0)
        g_wait(1)
        s_wait(n_items - 3, 1)
        transpose(1)
        s_start(n_items - 1, 1)
        s_wait(n_items - 2, 0)
        s_wait(n_items - 1, 1)

    return k


def kernel(token_indices, token_embedding_table):
    b, t = token_indices.shape
    v, d = token_embedding_table.shape
    idx_t = jnp.pad(token_indices.astype(jnp.int32).T, ((0, 6), (0, 0)))
    table_pad = jnp.pad(token_embedding_table, ((0, 0), (0, 24)))
    x = _make_kernel(b, t, v, d)(table_pad, idx_t)
    return jnp.transpose(x, (2, 0, 1))


# speedup vs baseline: 6.0525x; 3.4535x over previous
import functools

import jax
import jax.numpy as jnp
from jax import lax
from jax.experimental import pallas as pl
from jax.experimental.pallas import tpu as pltpu
from jax.experimental.pallas import tpu_sc as plsc

NUM_CORES = 2
NUM_SUBCORES = 16
LANES = 16
BLK = 128


def _make_kernel(batch, seq, vocab, dim):
    dpad = (dim + BLK - 1) // BLK * BLK
    spad = (seq + 7) // 8 * 8
    n_bb = batch // BLK
    n_vb = dpad // BLK
    assert n_bb * n_vb == 64
    tail = dim - (n_vb - 1) * BLK
    n_items = 2 * seq

    mesh = plsc.VectorSubcoreMesh(core_axis_name="c", subcore_axis_name="s")

    @functools.partial(
        pl.kernel,
        mesh=mesh,
        compiler_params=pltpu.CompilerParams(
            use_tc_tiling_on_sc=True, needs_layout_passes=False),
        out_type=jax.ShapeDtypeStruct((seq, dim, batch), jnp.float32),
        scratch_types=[
            pltpu.VMEM((spad, BLK), jnp.int32),
            pltpu.VMEM((BLK, BLK), jnp.float32),
            pltpu.VMEM((BLK, BLK), jnp.float32),
            pltpu.VMEM((BLK, BLK), jnp.float32),
            pltpu.VMEM((BLK, BLK), jnp.float32),
            pltpu.SemaphoreType.DMA,
            pltpu.SemaphoreType.DMA,
            pltpu.SemaphoreType.DMA,
            pltpu.SemaphoreType.DMA,
        ],
    )
    def k(table_hbm, idxT_hbm, out_hbm, idx_v, G0, G1, X0, X1, g0, g1, s0, s1):
        wid = lax.axis_index("s") * NUM_CORES + lax.axis_index("c")
        bb = wid // 4
        vb0 = 2 * (wid % 4)
        bcol = pl.multiple_of(bb * BLK, BLK)
        G = (G0, G1)
        XT = (X0, X1)
        gsem = (g0, g1)
        ssem = (s0, s1)

        pltpu.sync_copy(idxT_hbm.at[:, pl.ds(bcol, BLK)], idx_v)

        def tv(i):
            return vb0 + i // seq, i % seq

        def g_start(i, b):
            vb, t = tv(i)
            vcol = pl.multiple_of(vb * BLK, BLK)
            pltpu.async_copy(
                table_hbm.at[idx_v.at[t], pl.ds(vcol, BLK)], G[b], gsem[b])

        def g_wait(b):
            pltpu.make_async_copy(
                table_hbm.at[pl.ds(0, BLK), pl.ds(0, BLK)], G[b], gsem[b]).wait()

        def s_start(i, b):
            vb, t = tv(i)
            last = vb == (n_vb - 1)

            @pl.when(last)
            def _():
                pltpu.async_copy(
                    XT[b].at[pl.ds(0, tail)],
                    out_hbm.at[t, pl.ds((n_vb - 1) * BLK, tail), pl.ds(bcol, BLK)],
                    ssem[b])

            @pl.when(jnp.logical_not(last))
            def _():
                vcol = pl.multiple_of(vb * BLK, BLK)
                pltpu.async_copy(
                    XT[b], out_hbm.at[t, pl.ds(vcol, BLK), pl.ds(bcol, BLK)],
                    ssem[b])

        def s_wait(i, b):
            vb, _ = tv(i)
            last = vb == (n_vb - 1)

            @pl.when(last)
            def _():
                pltpu.make_async_copy(
                    XT[b].at[pl.ds(0, tail)],
                    out_hbm.at[0, pl.ds(0, tail), pl.ds(0, BLK)], ssem[b]).wait()

            @pl.when(jnp.logical_not(last))
            def _():
                pltpu.make_async_copy(
                    XT[b], out_hbm.at[0, pl.ds(0, BLK), pl.ds(0, BLK)],
                    ssem[b]).wait()

        iota = lax.iota(jnp.int32, LANES)
        colperm = [(iota + d) & (LANES - 1) for d in range(LANES)]

        def transpose(b):
            src = G[b]
            dst = XT[b]

            @plsc.parallel_loop(0, (BLK // LANES) ** 2, unroll=2)
            def _(i):
                rows = iota + lax.div(i, 8) * LANES
                v0 = lax.rem(i, 8) * LANES
                for d in range(LANES):
                    cols = colperm[d] + v0
                    vals = plsc.load_gather(src, [rows, cols])
                    plsc.store_scatter(dst, [cols, rows], vals)

        g_start(0, 0)

        @pl.loop(0, n_items, step=2)
        def _(i):
            g_wait(0)

            @pl.when(i >= 2)
            def _():
                s_wait(i - 2, 0)

            g_start(i + 1, 1)
            transpose(0)
            s_start(i, 0)
            g_wait(1)

            @pl.when(i >= 2)
            def _():
                s_wait(i - 1, 1)

            @pl.when(i + 2 < n_items)
            def _():
                g_start(i + 2, 0)

            transpose(1)
            s_start(i + 1, 1)

        s_wait(n_items - 2, 0)
        s_wait(n_items - 1, 1)

    return k


def kernel(token_indices, token_embedding_table):
    b, t = token_indices.shape
    v, d = token_embedding_table.shape
    idx_t = jnp.pad(token_indices.astype(jnp.int32).T, ((0, 6), (0, 0)))
    table_pad = jnp.pad(token_embedding_table, ((0, 0), (0, 24)))
    x = _make_kernel(b, t, v, d)(table_pad, idx_t)
    return jnp.transpose(x, (2, 0, 1))


# speedup vs baseline: 6.0823x; 1.0049x over previous
import functools

import jax
import jax.numpy as jnp
from jax import lax
from jax.experimental import pallas as pl
from jax.experimental.pallas import tpu as pltpu
from jax.experimental.pallas import tpu_sc as plsc

NUM_CORES = 2
NUM_SUBCORES = 16
LANES = 16
BLK = 128


def _make_kernel(batch, seq, vocab, dim):
    dpad = (dim + BLK - 1) // BLK * BLK
    spad = (seq + 7) // 8 * 8
    n_bb = batch // BLK
    n_vb = dpad // BLK
    assert n_bb * n_vb == 64
    tail = dim - (n_vb - 1) * BLK
    n_items = 2 * seq

    mesh = plsc.VectorSubcoreMesh(core_axis_name="c", subcore_axis_name="s")

    @functools.partial(
        pl.kernel,
        mesh=mesh,
        compiler_params=pltpu.CompilerParams(
            use_tc_tiling_on_sc=True, needs_layout_passes=False),
        out_type=jax.ShapeDtypeStruct((seq, dim, batch), jnp.float32),
        scratch_types=[
            pltpu.VMEM((spad, BLK), jnp.int32),
            pltpu.VMEM((BLK, BLK), jnp.float32),
            pltpu.VMEM((BLK, BLK), jnp.float32),
            pltpu.VMEM((BLK, BLK), jnp.float32),
            pltpu.VMEM((BLK, BLK), jnp.float32),
            pltpu.SemaphoreType.DMA,
            pltpu.SemaphoreType.DMA,
            pltpu.SemaphoreType.DMA,
            pltpu.SemaphoreType.DMA,
        ],
    )
    def k(table_hbm, idxT_hbm, out_hbm, idx_v, G0, G1, X0, X1, g0, g1, s0, s1):
        wid = lax.axis_index("s") * NUM_CORES + lax.axis_index("c")
        bb = wid // 4
        vb0 = 2 * (wid % 4)
        bcol = pl.multiple_of(bb * BLK, BLK)
        G = (G0, G1)
        XT = (X0, X1)
        gsem = (g0, g1)
        ssem = (s0, s1)

        pltpu.sync_copy(idxT_hbm.at[:, pl.ds(bcol, BLK)], idx_v)

        def tv(i):
            return vb0 + i // seq, i % seq

        def g_start(i, b):
            vb, t = tv(i)
            vcol = pl.multiple_of(vb * BLK, BLK)
            pltpu.async_copy(
                table_hbm.at[idx_v.at[t], pl.ds(vcol, BLK)], G[b], gsem[b])

        def g_wait(b):
            pltpu.make_async_copy(
                table_hbm.at[pl.ds(0, BLK), pl.ds(0, BLK)], G[b], gsem[b]).wait()

        def s_start(i, b):
            vb, t = tv(i)
            last = vb == (n_vb - 1)

            @pl.when(last)
            def _():
                pltpu.async_copy(
                    XT[b].at[pl.ds(0, tail)],
                    out_hbm.at[t, pl.ds((n_vb - 1) * BLK, tail), pl.ds(bcol, BLK)],
                    ssem[b])

            @pl.when(jnp.logical_not(last))
            def _():
                vcol = pl.multiple_of(vb * BLK, BLK)
                pltpu.async_copy(
                    XT[b], out_hbm.at[t, pl.ds(vcol, BLK), pl.ds(bcol, BLK)],
                    ssem[b])

        def s_wait(i, b):
            vb, _ = tv(i)
            last = vb == (n_vb - 1)

            @pl.when(last)
            def _():
                pltpu.make_async_copy(
                    XT[b].at[pl.ds(0, tail)],
                    out_hbm.at[0, pl.ds(0, tail), pl.ds(0, BLK)], ssem[b]).wait()

            @pl.when(jnp.logical_not(last))
            def _():
                pltpu.make_async_copy(
                    XT[b], out_hbm.at[0, pl.ds(0, BLK), pl.ds(0, BLK)],
                    ssem[b]).wait()

        iota = lax.iota(jnp.int32, LANES)
        colperm = [(iota + d) & (LANES - 1) for d in range(LANES)]

        def transpose(b):
            src = G[b]
            dst = XT[b]

            @plsc.parallel_loop(0, (BLK // LANES) ** 2, unroll=4)
            def _(i):
                rows = iota + lax.div(i, 8) * LANES
                v0 = lax.rem(i, 8) * LANES
                for d in range(LANES):
                    cols = colperm[d] + v0
                    vals = plsc.load_gather(src, [rows, cols])
                    plsc.store_scatter(dst, [cols, rows], vals)

        g_start(0, 0)

        @pl.loop(0, n_items, step=2)
        def _(i):
            g_wait(0)

            @pl.when(i >= 2)
            def _():
                s_wait(i - 2, 0)

            g_start(i + 1, 1)
            transpose(0)
            s_start(i, 0)
            g_wait(1)

            @pl.when(i >= 2)
            def _():
                s_wait(i - 1, 1)

            @pl.when(i + 2 < n_items)
            def _():
                g_start(i + 2, 0)

            transpose(1)
            s_start(i + 1, 1)

        s_wait(n_items - 2, 0)
        s_wait(n_items - 1, 1)

    return k


def kernel(token_indices, token_embedding_table):
    b, t = token_indices.shape
    v, d = token_embedding_table.shape
    idx_t = jnp.pad(token_indices.astype(jnp.int32).T, ((0, 6), (0, 0)))
    table_pad = jnp.pad(token_embedding_table, ((0, 0), (0, 24)))
    x = _make_kernel(b, t, v, d)(table_pad, idx_t)
    return jnp.transpose(x, (2, 0, 1))


# speedup vs baseline: 7.2598x; 1.1936x over previous
import functools

import jax
import jax.numpy as jnp
from jax import lax
from jax.experimental import pallas as pl
from jax.experimental.pallas import tpu as pltpu
from jax.experimental.pallas import tpu_sc as plsc

NUM_CORES = 2
NUM_SUBCORES = 16
LANES = 16
BLK = 128


def _make_kernel(batch, seq, vocab, dim):
    dpad = (dim + BLK - 1) // BLK * BLK
    spad = (seq + 7) // 8 * 8
    n_bb = batch // BLK
    n_vb = dpad // BLK
    assert n_bb * n_vb == 64
    tail = dim - (n_vb - 1) * BLK

    mesh = plsc.VectorSubcoreMesh(core_axis_name="c", subcore_axis_name="s")

    @functools.partial(
        pl.kernel,
        mesh=mesh,
        compiler_params=pltpu.CompilerParams(
            use_tc_tiling_on_sc=True, needs_layout_passes=False),
        out_type=jax.ShapeDtypeStruct((seq, dim, batch), jnp.float32),
        scratch_types=[
            pltpu.VMEM((spad, BLK), jnp.int32),
            pltpu.VMEM((BLK, 2 * BLK), jnp.float32),
            pltpu.VMEM((BLK, 2 * BLK), jnp.float32),
            pltpu.VMEM((BLK, BLK), jnp.float32),
            pltpu.VMEM((BLK, BLK), jnp.float32),
            pltpu.SemaphoreType.DMA,
            pltpu.SemaphoreType.DMA,
            pltpu.SemaphoreType.DMA,
            pltpu.SemaphoreType.DMA,
        ],
    )
    def k(table_hbm, idxT_hbm, out_hbm, idx_v, G0, G1, X0, X1, g0, g1, s0, s1):
        wid = lax.axis_index("s") * NUM_CORES + lax.axis_index("c")
        bb = wid // 4
        vb0 = 2 * (wid % 4)
        bcol = pl.multiple_of(bb * BLK, BLK)
        last = vb0 + 1 == n_vb - 1
        G = (G0, G1)
        XT = (X0, X1)
        gsem = (g0, g1)
        ssem = (s0, s1)

        pltpu.sync_copy(idxT_hbm.at[:, pl.ds(bcol, BLK)], idx_v)

        def g_start(t, b):
            vcol = pl.multiple_of(vb0 * BLK, BLK)
            pltpu.async_copy(
                table_hbm.at[idx_v.at[t], pl.ds(vcol, 2 * BLK)], G[b], gsem[b])

        def g_wait(b):
            pltpu.make_async_copy(
                table_hbm.at[pl.ds(0, BLK), pl.ds(0, 2 * BLK)], G[b],
                gsem[b]).wait()

        def s_start0(t):
            vcol = pl.multiple_of(vb0 * BLK, BLK)
            pltpu.async_copy(
                XT[0], out_hbm.at[t, pl.ds(vcol, BLK), pl.ds(bcol, BLK)], ssem[0])

        def s_wait0():
            pltpu.make_async_copy(
                XT[0], out_hbm.at[0, pl.ds(0, BLK), pl.ds(0, BLK)], ssem[0]).wait()

        def s_start1(t):
            @pl.when(last)
            def _():
                pltpu.async_copy(
                    XT[1].at[pl.ds(0, tail)],
                    out_hbm.at[t, pl.ds((n_vb - 1) * BLK, tail), pl.ds(bcol, BLK)],
                    ssem[1])

            @pl.when(jnp.logical_not(last))
            def _():
                vcol = pl.multiple_of((vb0 + 1) * BLK, BLK)
                pltpu.async_copy(
                    XT[1], out_hbm.at[t, pl.ds(vcol, BLK), pl.ds(bcol, BLK)],
                    ssem[1])

        def s_wait1():
            @pl.when(last)
            def _():
                pltpu.make_async_copy(
                    XT[1].at[pl.ds(0, tail)],
                    out_hbm.at[0, pl.ds(0, tail), pl.ds(0, BLK)], ssem[1]).wait()

            @pl.when(jnp.logical_not(last))
            def _():
                pltpu.make_async_copy(
                    XT[1], out_hbm.at[0, pl.ds(0, BLK), pl.ds(0, BLK)],
                    ssem[1]).wait()

        iota = lax.iota(jnp.int32, LANES)
        colperm = [(iota + d) & (LANES - 1) for d in range(LANES)]

        def transpose(b, h):
            src = G[b]
            dst = XT[h]

            @plsc.parallel_loop(0, (BLK // LANES) ** 2, unroll=4)
            def _(i):
                rows = iota + lax.div(i, 8) * LANES
                v0 = lax.rem(i, 8) * LANES + h * BLK
                for d in range(LANES):
                    cols = colperm[d] + v0
                    vals = plsc.load_gather(src, [rows, cols])
                    plsc.store_scatter(dst, [cols - h * BLK, rows], vals)

        g_start(0, 0)

        @pl.loop(0, seq, step=2)
        def _(t):
            for off in (0, 1):
                tt = t + off
                b = off
                g_wait(b)

                @pl.when(tt + 1 < seq)
                def _():
                    g_start(tt + 1, 1 - b)

                @pl.when(tt >= 1)
                def _():
                    s_wait0()

                transpose(b, 0)
                s_start0(tt)

                @pl.when(tt >= 1)
                def _():
                    s_wait1()

                transpose(b, 1)
                s_start1(tt)

        s_wait0()
        s_wait1()

    return k


def kernel(token_indices, token_embedding_table):
    b, t = token_indices.shape
    v, d = token_embedding_table.shape
    idx_t = jnp.pad(token_indices.astype(jnp.int32).T, ((0, 6), (0, 0)))
    table_pad = jnp.pad(token_embedding_table, ((0, 0), (0, 24)))
    x = _make_kernel(b, t, v, d)(table_pad, idx_t)
    return jnp.transpose(x, (2, 0, 1))
